# baseline (speedup 1.0000x reference)
import functools

import jax
import jax.numpy as jnp
from jax.experimental import pallas as pl
from jax.experimental.pallas import tpu as pltpu

_B, _S, _D_MODEL = 4, 2048, 1024
_N_IN, _N_PROC, _N_OUT, _D_PV = 4096, 2048, 4096, 512
_K_IN, _K_PROC, _K_OUT = _N_IN // 8, _N_PROC // 8, _N_OUT // 8


def _gelu(v):
    return 0.5 * v * (1.0 + jax.lax.erf(v * 0.7071067811865476))


def _kth_largest(acts, k):
    bits = jax.lax.bitcast_convert_type(acts, jnp.uint32)
    top = jnp.uint32(0x80000000)
    codes = jnp.where(bits >= top, ~bits, bits | top)

    def body(i, res):
        cand = res | (jnp.uint32(1) << (jnp.uint32(31) - i.astype(jnp.uint32)))
        cnt = jnp.sum((codes >= cand).astype(jnp.int32), axis=1, keepdims=True)
        return jnp.where(cnt >= k, cand, res)

    res = jax.lax.fori_loop(0, 32, body, jnp.zeros((acts.shape[0], 1), jnp.uint32))
    thr_bits = jnp.where(res >= top, res ^ top, ~res)
    return jax.lax.bitcast_convert_type(thr_bits, jnp.float32)


_TS1 = 512
_NS1 = _B * (_S // _TS1)
_TPF = 256
_NS2 = _N_PROC // _TPF


def _stage12_body(x_ref, w_ref, wp_ref, out_ref, acts_ref):
    i = pl.program_id(0)

    @pl.when(i == 0)
    def _():
        acts_ref[...] = jnp.zeros_like(acts_ref)

    @pl.when(i < _NS1)
    def _():
        b = i // (_S // _TS1)
        scores = jax.lax.dot_general(
            x_ref[0], w_ref[...], (((1,), (1,)), ((), ())),
            preferred_element_type=jnp.float32)
        partial = jnp.sum(_gelu(scores), axis=0, keepdims=True) * (1.0 / _S)
        rows = jax.lax.broadcasted_iota(jnp.int32, (_B, 1), 0)
        acts_ref[...] += jnp.where(rows == b, partial, 0.0)

    @pl.when(i == _NS1)
    def _():
        acts = acts_ref[...]
        thr = _kth_largest(acts, _K_IN)
        acts_ref[...] = jnp.where(acts >= thr, acts, 0.0)

    @pl.when(i >= _NS1)
    def _():
        pscores = jax.lax.dot_general(
            acts_ref[...], wp_ref[...], (((1,), (1,)), ((), ())),
            preferred_element_type=jnp.float32)
        out_ref[...] = _gelu(pscores)


def _stage12(x, input_patterns, process_input_weights):
    nsteps = _NS1 + _NS2
    sb = _S // _TS1

    def x_map(i):
        j = jnp.minimum(i, _NS1 - 1)
        return (j // sb, j % sb, 0)

    return pl.pallas_call(
        _stage12_body,
        grid=(nsteps,),
        in_specs=[
            pl.BlockSpec((1, _TS1, _D_MODEL), x_map),
            pl.BlockSpec((_N_IN, _D_MODEL), lambda i: (0, 0)),
            pl.BlockSpec((_TPF, _N_IN),
                         lambda i: (jnp.clip(i - _NS1, 0, _NS2 - 1), 0)),
        ],
        out_specs=pl.BlockSpec(
            (_B, _TPF), lambda i: (0, jnp.clip(i - _NS1, 0, _NS2 - 1))),
        out_shape=jax.ShapeDtypeStruct((_B, _N_PROC), jnp.float32),
        scratch_shapes=[pltpu.VMEM((_B, _N_IN), jnp.float32)],
        compiler_params=pltpu.CompilerParams(
            dimension_semantics=("arbitrary",)),
    )(x, input_patterns, process_input_weights)


def _tail_body(pacts_ref, pv_ref, wo_ref, p_ref, out_ref, macts_ref):
    @pl.when(pl.program_id(0) == 0)
    def _():
        pacts = pacts_ref[...]
        thr = _kth_largest(pacts, _K_PROC)
        mask = pacts >= thr
        rowmax = jnp.max(pacts, axis=1, keepdims=True)
        e = jnp.where(mask, jnp.exp(pacts - rowmax), 0.0)
        w = e / jnp.sum(e, axis=1, keepdims=True)
        agg = jax.lax.dot_general(
            w, pv_ref[...], (((1,), (0,)), ((), ())),
            preferred_element_type=jnp.float32)
        oscores = jax.lax.dot_general(
            agg, wo_ref[...], (((1,), (1,)), ((), ())),
            preferred_element_type=jnp.float32)
        oacts = _gelu(oscores)
        thr2 = _kth_largest(oacts, _K_OUT)
        macts_ref[...] = jnp.where(oacts >= thr2, oacts, 0.0)

    out_ref[...] = jax.lax.dot_general(
        macts_ref[...], p_ref[...], (((1,), (0,)), ((), ())),
        preferred_element_type=jnp.float32)


def _tail(pacts, process_values, output_input_weights, output_patterns):
    TD = 512
    return pl.pallas_call(
        _tail_body,
        grid=(_D_MODEL // TD,),
        in_specs=[
            pl.BlockSpec((_B, _N_PROC), lambda d: (0, 0)),
            pl.BlockSpec((_N_PROC, _D_PV), lambda d: (0, 0)),
            pl.BlockSpec((_N_OUT, _D_PV), lambda d: (0, 0)),
            pl.BlockSpec((_N_OUT, TD), lambda d: (0, d)),
        ],
        out_specs=pl.BlockSpec((_B, TD), lambda d: (0, d)),
        out_shape=jax.ShapeDtypeStruct((_B, _D_MODEL), jnp.float32),
        scratch_shapes=[pltpu.VMEM((_B, _N_OUT), jnp.float32)],
        compiler_params=pltpu.CompilerParams(
            dimension_semantics=("arbitrary",)),
    )(pacts, process_values, output_input_weights, output_patterns)


def kernel(x, input_patterns, process_input_weights, process_values,
           output_input_weights, output_patterns):
    pacts = _stage12(x, input_patterns, process_input_weights)
    out_row = _tail(pacts, process_values, output_input_weights,
                    output_patterns)
    return jnp.broadcast_to(out_row[:, None, :], (_B, _S, _D_MODEL))


# speedup vs baseline: 1.0268x; 1.0268x over previous
import functools

import jax
import jax.numpy as jnp
from jax.experimental import pallas as pl
from jax.experimental.pallas import tpu as pltpu

_B, _S, _D_MODEL = 4, 2048, 1024
_N_IN, _N_PROC, _N_OUT, _D_PV = 4096, 2048, 4096, 512
_K_IN, _K_PROC, _K_OUT = _N_IN // 8, _N_PROC // 8, _N_OUT // 8


def _gelu(v):
    return 0.5 * v * (1.0 + jax.lax.erf(v * 0.7071067811865476))


def _kth_largest(acts, k):
    bits = jax.lax.bitcast_convert_type(acts, jnp.uint32)
    top = jnp.uint32(0x80000000)
    codes = jnp.where(bits >= top, ~bits, bits | top)

    def body(i, res):
        cand = res | (jnp.uint32(1) << (jnp.uint32(31) - i.astype(jnp.uint32)))
        cnt = jnp.sum((codes >= cand).astype(jnp.int32), axis=1, keepdims=True)
        return jnp.where(cnt >= k, cand, res)

    res = jax.lax.fori_loop(0, 32, body, jnp.zeros((acts.shape[0], 1), jnp.uint32))
    thr_bits = jnp.where(res >= top, res ^ top, ~res)
    return jax.lax.bitcast_convert_type(thr_bits, jnp.float32)


_TS1 = 2048
_NS1 = _B * (_S // _TS1)
_TPF = 256
_NS2 = _N_PROC // _TPF


def _stage12_body(x_ref, w_ref, wp_ref, out_ref, acts_ref):
    i = pl.program_id(0)

    @pl.when(i == 0)
    def _():
        acts_ref[...] = jnp.zeros_like(acts_ref)

    @pl.when(i < _NS1)
    def _():
        b = i // (_S // _TS1)
        scores = jax.lax.dot_general(
            x_ref[0], w_ref[...], (((1,), (1,)), ((), ())),
            preferred_element_type=jnp.float32)
        partial = jnp.sum(_gelu(scores), axis=0, keepdims=True) * (1.0 / _S)
        rows = jax.lax.broadcasted_iota(jnp.int32, (_B, 1), 0)
        acts_ref[...] += jnp.where(rows == b, partial, 0.0)

    @pl.when(i == _NS1)
    def _():
        acts = acts_ref[...]
        thr = _kth_largest(acts, _K_IN)
        acts_ref[...] = jnp.where(acts >= thr, acts, 0.0)

    @pl.when(i >= _NS1)
    def _():
        pscores = jax.lax.dot_general(
            acts_ref[...], wp_ref[...], (((1,), (1,)), ((), ())),
            preferred_element_type=jnp.float32)
        out_ref[...] = _gelu(pscores)


def _stage12(x, input_patterns, process_input_weights):
    nsteps = _NS1 + _NS2
    sb = _S // _TS1

    def x_map(i):
        j = jnp.minimum(i, _NS1 - 1)
        return (j // sb, j % sb, 0)

    return pl.pallas_call(
        _stage12_body,
        grid=(nsteps,),
        in_specs=[
            pl.BlockSpec((1, _TS1, _D_MODEL), x_map),
            pl.BlockSpec((_N_IN, _D_MODEL), lambda i: (0, 0)),
            pl.BlockSpec((_TPF, _N_IN),
                         lambda i: (jnp.clip(i - _NS1, 0, _NS2 - 1), 0)),
        ],
        out_specs=pl.BlockSpec(
            (_B, _TPF), lambda i: (0, jnp.clip(i - _NS1, 0, _NS2 - 1))),
        out_shape=jax.ShapeDtypeStruct((_B, _N_PROC), jnp.float32),
        scratch_shapes=[pltpu.VMEM((_B, _N_IN), jnp.float32)],
        compiler_params=pltpu.CompilerParams(
            dimension_semantics=("arbitrary",)),
    )(x, input_patterns, process_input_weights)


def _tail_body(pacts_ref, pv_ref, wo_ref, p_ref, out_ref, macts_ref):
    @pl.when(pl.program_id(0) == 0)
    def _():
        pacts = pacts_ref[...]
        thr = _kth_largest(pacts, _K_PROC)
        mask = pacts >= thr
        rowmax = jnp.max(pacts, axis=1, keepdims=True)
        e = jnp.where(mask, jnp.exp(pacts - rowmax), 0.0)
        w = e / jnp.sum(e, axis=1, keepdims=True)
        agg = jax.lax.dot_general(
            w, pv_ref[...], (((1,), (0,)), ((), ())),
            preferred_element_type=jnp.float32)
        oscores = jax.lax.dot_general(
            agg, wo_ref[...], (((1,), (1,)), ((), ())),
            preferred_element_type=jnp.float32)
        oacts = _gelu(oscores)
        thr2 = _kth_largest(oacts, _K_OUT)
        macts_ref[...] = jnp.where(oacts >= thr2, oacts, 0.0)

    out_ref[...] = jax.lax.dot_general(
        macts_ref[...], p_ref[...], (((1,), (0,)), ((), ())),
        preferred_element_type=jnp.float32)


def _tail(pacts, process_values, output_input_weights, output_patterns):
    TD = 512
    return pl.pallas_call(
        _tail_body,
        grid=(_D_MODEL // TD,),
        in_specs=[
            pl.BlockSpec((_B, _N_PROC), lambda d: (0, 0)),
            pl.BlockSpec((_N_PROC, _D_PV), lambda d: (0, 0)),
            pl.BlockSpec((_N_OUT, _D_PV), lambda d: (0, 0)),
            pl.BlockSpec((_N_OUT, TD), lambda d: (0, d)),
        ],
        out_specs=pl.BlockSpec((_B, TD), lambda d: (0, d)),
        out_shape=jax.ShapeDtypeStruct((_B, _D_MODEL), jnp.float32),
        scratch_shapes=[pltpu.VMEM((_B, _N_OUT), jnp.float32)],
        compiler_params=pltpu.CompilerParams(
            dimension_semantics=("arbitrary",)),
    )(pacts, process_values, output_input_weights, output_patterns)


def kernel(x, input_patterns, process_input_weights, process_values,
           output_input_weights, output_patterns):
    pacts = _stage12(x, input_patterns, process_input_weights)
    out_row = _tail(pacts, process_values, output_input_weights,
                    output_patterns)
    return jnp.broadcast_to(out_row[:, None, :], (_B, _S, _D_MODEL))


# speedup vs baseline: 1.0852x; 1.0569x over previous
import functools

import jax
import jax.numpy as jnp
from jax.experimental import pallas as pl
from jax.experimental.pallas import tpu as pltpu

_B, _S, _D_MODEL = 4, 2048, 1024
_N_IN, _N_PROC, _N_OUT, _D_PV = 4096, 2048, 4096, 512
_K_IN, _K_PROC, _K_OUT = _N_IN // 8, _N_PROC // 8, _N_OUT // 8


def _gelu(v):
    return 0.5 * v * (1.0 + jax.lax.erf(v * 0.7071067811865476))


def _kth_largest(acts, k):
    bits = jax.lax.bitcast_convert_type(acts, jnp.uint32)
    top = jnp.uint32(0x80000000)
    codes = jnp.where(bits >= top, ~bits, bits | top)

    def body(i, res):
        cand = res | (jnp.uint32(1) << (jnp.uint32(31) - i.astype(jnp.uint32)))
        cnt = jnp.sum((codes >= cand).astype(jnp.int32), axis=1, keepdims=True)
        return jnp.where(cnt >= k, cand, res)

    res = jax.lax.fori_loop(0, 32, body, jnp.zeros((acts.shape[0], 1), jnp.uint32))
    thr_bits = jnp.where(res >= top, res ^ top, ~res)
    return jax.lax.bitcast_convert_type(thr_bits, jnp.float32)


_TS1 = 1024
_NS1 = _B * (_S // _TS1)
_TPF = 512
_NS2 = _N_PROC // _TPF


def _stage12_body(x_ref, w_ref, wp_ref, out_ref, acts_ref):
    i = pl.program_id(0)

    @pl.when(i == 0)
    def _():
        acts_ref[...] = jnp.zeros_like(acts_ref)

    @pl.when(i < _NS1)
    def _():
        b = i // (_S // _TS1)
        scores = jax.lax.dot_general(
            x_ref[0], w_ref[...], (((1,), (1,)), ((), ())),
            preferred_element_type=jnp.float32)
        partial = jnp.sum(_gelu(scores), axis=0, keepdims=True) * (1.0 / _S)
        rows = jax.lax.broadcasted_iota(jnp.int32, (_B, 1), 0)
        acts_ref[...] += jnp.where(rows == b, partial, 0.0)

    @pl.when(i == _NS1)
    def _():
        acts = acts_ref[...]
        thr = _kth_largest(acts, _K_IN)
        acts_ref[...] = jnp.where(acts >= thr, acts, 0.0)

    @pl.when(i >= _NS1)
    def _():
        pscores = jax.lax.dot_general(
            acts_ref[...], wp_ref[...], (((1,), (1,)), ((), ())),
            preferred_element_type=jnp.float32)
        out_ref[...] = _gelu(pscores)


def _stage12(x, input_patterns, process_input_weights):
    nsteps = _NS1 + _NS2
    sb = _S // _TS1

    def x_map(i):
        j = jnp.minimum(i, _NS1 - 1)
        return (j // sb, j % sb, 0)

    return pl.pallas_call(
        _stage12_body,
        grid=(nsteps,),
        in_specs=[
            pl.BlockSpec((1, _TS1, _D_MODEL), x_map),
            pl.BlockSpec((_N_IN, _D_MODEL), lambda i: (0, 0)),
            pl.BlockSpec((_TPF, _N_IN),
                         lambda i: (jnp.clip(i - _NS1, 0, _NS2 - 1), 0)),
        ],
        out_specs=pl.BlockSpec(
            (_B, _TPF), lambda i: (0, jnp.clip(i - _NS1, 0, _NS2 - 1))),
        out_shape=jax.ShapeDtypeStruct((_B, _N_PROC), jnp.float32),
        scratch_shapes=[pltpu.VMEM((_B, _N_IN), jnp.float32)],
        compiler_params=pltpu.CompilerParams(
            dimension_semantics=("arbitrary",)),
    )(x, input_patterns, process_input_weights)


def _tail_body(pacts_ref, pv_ref, wo_ref, p_ref, out_ref, macts_ref):
    @pl.when(pl.program_id(0) == 0)
    def _():
        pacts = pacts_ref[...]
        thr = _kth_largest(pacts, _K_PROC)
        mask = pacts >= thr
        rowmax = jnp.max(pacts, axis=1, keepdims=True)
        e = jnp.where(mask, jnp.exp(pacts - rowmax), 0.0)
        w = e / jnp.sum(e, axis=1, keepdims=True)
        agg = jax.lax.dot_general(
            w, pv_ref[...], (((1,), (0,)), ((), ())),
            preferred_element_type=jnp.float32)
        oscores = jax.lax.dot_general(
            agg, wo_ref[...], (((1,), (1,)), ((), ())),
            preferred_element_type=jnp.float32)
        oacts = _gelu(oscores)
        thr2 = _kth_largest(oacts, _K_OUT)
        macts_ref[...] = jnp.where(oacts >= thr2, oacts, 0.0)

    out_ref[...] = jax.lax.dot_general(
        macts_ref[...], p_ref[...], (((1,), (0,)), ((), ())),
        preferred_element_type=jnp.float32)


def _tail(pacts, process_values, output_input_weights, output_patterns):
    TD = 512
    return pl.pallas_call(
        _tail_body,
        grid=(_D_MODEL // TD,),
        in_specs=[
            pl.BlockSpec((_B, _N_PROC), lambda d: (0, 0)),
            pl.BlockSpec((_N_PROC, _D_PV), lambda d: (0, 0)),
            pl.BlockSpec((_N_OUT, _D_PV), lambda d: (0, 0)),
            pl.BlockSpec((_N_OUT, TD), lambda d: (0, d)),
        ],
        out_specs=pl.BlockSpec((_B, TD), lambda d: (0, d)),
        out_shape=jax.ShapeDtypeStruct((_B, _D_MODEL), jnp.float32),
        scratch_shapes=[pltpu.VMEM((_B, _N_OUT), jnp.float32)],
        compiler_params=pltpu.CompilerParams(
            dimension_semantics=("arbitrary",)),
    )(pacts, process_values, output_input_weights, output_patterns)


def kernel(x, input_patterns, process_input_weights, process_values,
           output_input_weights, output_patterns):
    pacts = _stage12(x, input_patterns, process_input_weights)
    out_row = _tail(pacts, process_values, output_input_weights,
                    output_patterns)
    return jnp.broadcast_to(out_row[:, None, :], (_B, _S, _D_MODEL))
